# baseline (speedup 1.0000x reference)
import functools

import numpy as np
import jax
import jax.numpy as jnp
from jax import lax
from jax.experimental import pallas as pl
from jax.experimental.pallas import tpu as pltpu
from jax.experimental.pallas import tpu_sc as plsc

BATCH = 512
IN_SIZE = 3072
NEURONS = 16000
NUM_CLASSES = 10
TAU = 100.0

NC, NS = 2, 16
BLANES = 32
NW = NC * NS
NPW = NEURONS // NW
CHUNK = 10
NCHUNK = NPW // CHUNK
HB = BATCH // 2
WSLICES = HB // BLANES

_COEF_MAT = np.zeros((4, 16), np.float32)
_COEF_MAT[0, 8:16] = 1.0
_COEF_MAT[1, [2, 3, 6, 7]] = 1.0
_COEF_MAT[1, [8, 9, 12, 13]] = -1.0
_COEF_MAT[2, [4, 5, 6, 7]] = 1.0
_COEF_MAT[2, [8, 9, 10, 11]] = -1.0
_COEF_MAT[3, [1, 8, 11, 13]] = 1.0
_COEF_MAT[3, [2, 4, 7, 14]] = -1.0
_COEF_MAT[3, 6] = -2.0
_COEF_MAT[3, 9] = 2.0

def _coef_body(b_ref, q_ref, ws_ref, out_ref):
    w = ws_ref[0]
    e = jnp.exp(w)
    s = jnp.dot(e, b_ref[...], preferred_element_type=jnp.float32)
    p = e / s
    out_ref[0] = jnp.dot(p, q_ref[...], preferred_element_type=jnp.float32)


def _coef_call(wf):
    rows = NEURONS // 8
    bseg = np.kron(np.eye(8, dtype=np.float32), np.ones((16, 16), np.float32))
    qmat = np.kron(np.eye(8, dtype=np.float32),
                   np.tile(_COEF_MAT, (4, 1)).T.astype(np.float32))
    return pl.pallas_call(
        _coef_body,
        grid=(4,),
        in_specs=[pl.BlockSpec((128, 128), lambda l: (0, 0)),
                  pl.BlockSpec((128, 128), lambda l: (0, 0)),
                  pl.BlockSpec((1, rows, 128), lambda l: (l, 0, 0))],
        out_specs=pl.BlockSpec((1, rows, 128), lambda l: (l, 0, 0)),
        out_shape=jax.ShapeDtypeStruct((4, rows, 128), jnp.float32),
    )(jnp.asarray(bseg), jnp.asarray(qmat), wf)


def _sc_layer_body(table, ia, ib, oi, coef, out,
                   idx_a, idx_b, idx_o,
                   a0, a1, b0, b1, cf0, cf1,
                   sem_a0, sem_a1, sem_b0, sem_b1, sem_s0, sem_s1,
                   sem_c0, sem_c1):
    wid = lax.axis_index("s") * NC + lax.axis_index("c")
    pltpu.sync_copy(ia.at[wid], idx_a)
    pltpu.sync_copy(ib.at[wid], idx_b)
    pltpu.sync_copy(oi.at[wid], idx_o)
    cw = coef.at[wid]

    abuf = (a0, a1)
    bbuf = (b0, b1)
    cbuf = (cf0, cf1)
    sema = (sem_a0, sem_a1)
    semb = (sem_b0, sem_b1)
    sems = (sem_s0, sem_s1)
    semc = (sem_c0, sem_c1)

    def issue_gather(k1, nxt):
        pltpu.async_copy(table.at[idx_a.at[k1]], abuf[nxt], sema[nxt])
        pltpu.async_copy(table.at[idx_b.at[k1]], bbuf[nxt], semb[nxt])
        pltpu.async_copy(cw.at[k1], cbuf[nxt], semc[nxt])

    def wait_gather(k, cur):
        pltpu.make_async_copy(table.at[idx_a.at[k]], abuf[cur], sema[cur]).wait()
        pltpu.make_async_copy(table.at[idx_b.at[k]], bbuf[cur], semb[cur]).wait()
        pltpu.make_async_copy(cw.at[k], cbuf[cur], semc[cur]).wait()

    def wait_store(k, buf):
        pltpu.make_async_copy(abuf[buf], out.at[idx_o.at[k]], sems[buf]).wait()

    def compute(k, cur):
        rav = abuf[cur].bitcast(jnp.bfloat16)
        rbv = bbuf[cur].bitcast(jnp.bfloat16)
        cfb = cbuf[cur]
        for g in range(CHUNK):
            c0 = cfb[g, pl.ds(0, BLANES)]
            ca = cfb[g, pl.ds(BLANES, BLANES)]
            cb = cfb[g, pl.ds(2 * BLANES, BLANES)]
            cab = cfb[g, pl.ds(3 * BLANES, BLANES)]
            for r in range(2 * g, 2 * g + 2):
                for s in range(WSLICES):
                    va = rav[r, pl.ds(s * BLANES, BLANES)]
                    vb = rbv[r, pl.ds(s * BLANES, BLANES)]
                    rav[r, pl.ds(s * BLANES, BLANES)] = (c0 + ca * va) + vb * (cb + cab * va)

    def stage(k, cur, first=False, last=False):
        nxt = 1 - cur
        if not last:
            if not first:
                wait_store(k, nxt)
            issue_gather(k + 1, nxt)
        wait_gather(k, cur)
        compute(k, cur)
        pltpu.async_copy(abuf[cur], out.at[idx_o.at[k]], sems[cur])

    issue_gather(0, 0)
    stage(0, 0, first=True)

    def mid(i, carry):
        stage(2 * i + 1, 1)
        stage(2 * i + 2, 0)
        return carry

    lax.fori_loop(0, (NCHUNK - 2) // 2, mid, 0, unroll=False)
    stage(NCHUNK - 1, 1, last=True)
    wait_store(NCHUNK - 2, 0)
    wait_store(NCHUNK - 1, 1)


def _sc_layer(table, ia3, ib3, oi3, coef):
    mesh = plsc.VectorSubcoreMesh(core_axis_name="c", subcore_axis_name="s",
                                  num_cores=NC, num_subcores=NS)
    f = pl.kernel(
        _sc_layer_body,
        out_type=jax.ShapeDtypeStruct((NEURONS, HB), jnp.float32),
        mesh=mesh,
        scratch_types=[
            pltpu.VMEM((NCHUNK, CHUNK), jnp.int32),
            pltpu.VMEM((NCHUNK, CHUNK), jnp.int32),
            pltpu.VMEM((NCHUNK, CHUNK), jnp.int32),
            pltpu.VMEM((CHUNK, HB), jnp.float32),
            pltpu.VMEM((CHUNK, HB), jnp.float32),
            pltpu.VMEM((CHUNK, HB), jnp.float32),
            pltpu.VMEM((CHUNK, HB), jnp.float32),
            pltpu.VMEM((CHUNK, 128), jnp.bfloat16),
            pltpu.VMEM((CHUNK, 128), jnp.bfloat16),
            pltpu.SemaphoreType.DMA,
            pltpu.SemaphoreType.DMA,
            pltpu.SemaphoreType.DMA,
            pltpu.SemaphoreType.DMA,
            pltpu.SemaphoreType.DMA,
            pltpu.SemaphoreType.DMA,
            pltpu.SemaphoreType.DMA,
            pltpu.SemaphoreType.DMA,
        ],
    )
    return f(table, ia3, ib3, oi3, coef)


def _gsum_body(h_ref, out_ref):
    u = lax.bitcast_convert_type(h_ref[...], jnp.uint32)
    lo = lax.bitcast_convert_type(u << 16, jnp.float32)
    hi = lax.bitcast_convert_type(u & jnp.uint32(0xFFFF0000), jnp.float32)
    s_lo = jnp.sum(lo, axis=0, keepdims=True) * (1.0 / TAU)
    s_hi = jnp.sum(hi, axis=0, keepdims=True) * (1.0 / TAU)
    out_ref[0] = jnp.concatenate([s_lo, s_hi], axis=0)


def _gsum_call(h):
    per = NEURONS // NUM_CLASSES
    return pl.pallas_call(
        _gsum_body,
        grid=(NUM_CLASSES,),
        in_specs=[pl.BlockSpec((per, HB), lambda i: (i, 0))],
        out_specs=pl.BlockSpec((1, 2, HB), lambda i: (i, 0, 0)),
        out_shape=jax.ShapeDtypeStruct((NUM_CLASSES, 2, HB), jnp.float32),
    )(h)


def kernel(x, w0, w1, w2, w3, ia0, ib0, ia1, ib1, ia2, ib2, ia3, ib3):
    t = x.reshape(BATCH, IN_SIZE).T
    xb = t.astype(jnp.bfloat16)
    plo = lax.bitcast_convert_type(xb[:, 0::2], jnp.uint16).astype(jnp.uint32)
    phi = lax.bitcast_convert_type(xb[:, 1::2], jnp.uint16).astype(jnp.uint32)
    t = lax.bitcast_convert_type(plo | (phi << 16), jnp.float32)
    ws = jnp.stack([w0, w1, w2, w3]).reshape(4, NEURONS // 8, 128)
    coefs = _coef_call(ws)
    c4 = coefs.reshape(4, NEURONS, 16)[:, :, :4].astype(jnp.bfloat16)
    csp = jnp.repeat(c4, BLANES, axis=2)
    csp = csp.reshape(4, NW, NCHUNK, CHUNK, 128)
    oi3d = jnp.arange(NEURONS, dtype=jnp.int32).reshape(NW, NCHUNK, CHUNK)
    for l, (ia, ib) in enumerate([(ia0, ib0), (ia1, ib1), (ia2, ib2), (ia3, ib3)]):
        ia3d = ia.reshape(NW, NCHUNK, CHUNK)
        ib3d = ib.reshape(NW, NCHUNK, CHUNK)
        t = _sc_layer(t, ia3d, ib3d, oi3d, csp[l])
    y = _gsum_call(t)
    y = jnp.moveaxis(y, 1, 2).reshape(NUM_CLASSES, BATCH)
    return y.T


# speedup vs baseline: 2.9583x; 2.9583x over previous
import functools

import numpy as np
import jax
import jax.numpy as jnp
from jax import lax
from jax.experimental import pallas as pl
from jax.experimental.pallas import tpu as pltpu
from jax.experimental.pallas import tpu_sc as plsc

BATCH = 512
IN_SIZE = 3072
NEURONS = 16000
NUM_CLASSES = 10
TAU = 100.0

NC, NS, LANES = 2, 16, 16
NW = NC * NS
NPW = NEURONS // NW
CHUNK = 50
NCHUNK = NPW // CHUNK
PSLICES = BATCH // LANES

_COEF_MAT = np.zeros((4, 16), np.float32)
_COEF_MAT[0, 8:16] = 1.0
_COEF_MAT[1, [2, 3, 6, 7]] = 1.0
_COEF_MAT[1, [8, 9, 12, 13]] = -1.0
_COEF_MAT[2, [4, 5, 6, 7]] = 1.0
_COEF_MAT[2, [8, 9, 10, 11]] = -1.0
_COEF_MAT[3, [1, 8, 11, 13]] = 1.0
_COEF_MAT[3, [2, 4, 7, 14]] = -1.0
_COEF_MAT[3, 6] = -2.0
_COEF_MAT[3, 9] = 2.0

_GATHER_DNUMS = lax.GatherDimensionNumbers(
    offset_dims=(), collapsed_slice_dims=(0,), start_index_map=(0,))


def _lane_splat(v, lane):
    idx = jnp.full((LANES, 1), lane, jnp.int32)
    return lax.gather(v, idx, _GATHER_DNUMS, slice_sizes=(1,),
                      mode=lax.GatherScatterMode.PROMISE_IN_BOUNDS)


def _coef_body(b_ref, q_ref, ws_ref, out_ref):
    w = ws_ref[0]
    e = jnp.exp(w)
    s = jnp.dot(e, b_ref[...], preferred_element_type=jnp.float32)
    p = e / s
    out_ref[0] = jnp.dot(p, q_ref[...], preferred_element_type=jnp.float32)


def _coef_call(wf):
    rows = NEURONS // 8
    bseg = np.kron(np.eye(8, dtype=np.float32), np.ones((16, 16), np.float32))
    qmat = np.kron(np.eye(8, dtype=np.float32),
                   np.tile(_COEF_MAT, (4, 1)).T.astype(np.float32))
    return pl.pallas_call(
        _coef_body,
        grid=(4,),
        in_specs=[pl.BlockSpec((128, 128), lambda l: (0, 0)),
                  pl.BlockSpec((128, 128), lambda l: (0, 0)),
                  pl.BlockSpec((1, rows, 128), lambda l: (l, 0, 0))],
        out_specs=pl.BlockSpec((1, rows, 128), lambda l: (l, 0, 0)),
        out_shape=jax.ShapeDtypeStruct((4, rows, 128), jnp.float32),
    )(jnp.asarray(bseg), jnp.asarray(qmat), wf)


def _sc_layer_body(table, ia, ib, oi, coef, out,
                   idx_a, idx_b, idx_o,
                   a0, a1, b0, b1, cbuf,
                   sem_a0, sem_a1, sem_b0, sem_b1, sem_s0, sem_s1):
    wid = lax.axis_index("s") * NC + lax.axis_index("c")
    pltpu.sync_copy(ia.at[wid], idx_a)
    pltpu.sync_copy(ib.at[wid], idx_b)
    pltpu.sync_copy(oi.at[wid], idx_o)
    pltpu.sync_copy(coef.at[wid], cbuf)

    abuf = (a0, a1)
    bbuf = (b0, b1)
    sema = (sem_a0, sem_a1)
    semb = (sem_b0, sem_b1)
    sems = (sem_s0, sem_s1)

    def issue_gather(k1, nxt):
        pltpu.async_copy(table.at[idx_a.at[k1]], abuf[nxt], sema[nxt])
        pltpu.async_copy(table.at[idx_b.at[k1]], bbuf[nxt], semb[nxt])

    def wait_gather(k, cur):
        pltpu.make_async_copy(table.at[idx_a.at[k]], abuf[cur], sema[cur]).wait()
        pltpu.make_async_copy(table.at[idx_b.at[k]], bbuf[cur], semb[cur]).wait()

    def wait_store(k, buf):
        pltpu.make_async_copy(abuf[buf], out.at[idx_o.at[k]], sems[buf]).wait()

    def compute(k, cur):
        ra, rb = abuf[cur], bbuf[cur]

        def nbody(g, c2):
            cv = cbuf[0, pl.ds((k * CHUNK + g) * 16, 16)]
            c0 = _lane_splat(cv, 0)
            ca = _lane_splat(cv, 1)
            cb = _lane_splat(cv, 2)
            cab = _lane_splat(cv, 3)
            for s in range(PSLICES):
                va = ra[g, pl.ds(s * LANES, LANES)]
                vb = rb[g, pl.ds(s * LANES, LANES)]
                ra[g, pl.ds(s * LANES, LANES)] = (c0 + ca * va) + vb * (cb + cab * va)
            return c2

        lax.fori_loop(0, CHUNK, nbody, 0, unroll=False)

    def stage(k, cur, first=False, last=False):
        nxt = 1 - cur
        if not last:
            if not first:
                wait_store(k, nxt)
            issue_gather(k + 1, nxt)
        wait_gather(k, cur)
        compute(k, cur)
        pltpu.async_copy(abuf[cur], out.at[idx_o.at[k]], sems[cur])

    issue_gather(0, 0)
    stage(0, 0, first=True)

    def mid(i, carry):
        stage(2 * i + 1, 1)
        stage(2 * i + 2, 0)
        return carry

    lax.fori_loop(0, (NCHUNK - 2) // 2, mid, 0, unroll=False)
    stage(NCHUNK - 1, 1, last=True)
    wait_store(NCHUNK - 2, 0)
    wait_store(NCHUNK - 1, 1)


def _sc_layer(table, ia3, ib3, oi3, coef):
    mesh = plsc.VectorSubcoreMesh(core_axis_name="c", subcore_axis_name="s",
                                  num_cores=NC, num_subcores=NS)
    f = pl.kernel(
        _sc_layer_body,
        out_type=jax.ShapeDtypeStruct((NEURONS, BATCH), jnp.float32),
        mesh=mesh,
        scratch_types=[
            pltpu.VMEM((NCHUNK, CHUNK), jnp.int32),
            pltpu.VMEM((NCHUNK, CHUNK), jnp.int32),
            pltpu.VMEM((NCHUNK, CHUNK), jnp.int32),
            pltpu.VMEM((CHUNK, BATCH), jnp.float32),
            pltpu.VMEM((CHUNK, BATCH), jnp.float32),
            pltpu.VMEM((CHUNK, BATCH), jnp.float32),
            pltpu.VMEM((CHUNK, BATCH), jnp.float32),
            pltpu.VMEM((1, 16 * NPW), jnp.float32),
            pltpu.SemaphoreType.DMA,
            pltpu.SemaphoreType.DMA,
            pltpu.SemaphoreType.DMA,
            pltpu.SemaphoreType.DMA,
            pltpu.SemaphoreType.DMA,
            pltpu.SemaphoreType.DMA,
        ],
    )
    return f(table, ia3, ib3, oi3, coef)


def _gsum_body(h_ref, out_ref):
    out_ref[0] = jnp.sum(h_ref[...], axis=0, keepdims=True) * (1.0 / TAU)


def _gsum_call(h):
    per = NEURONS // NUM_CLASSES
    return pl.pallas_call(
        _gsum_body,
        grid=(NUM_CLASSES,),
        in_specs=[pl.BlockSpec((per, BATCH), lambda i: (i, 0))],
        out_specs=pl.BlockSpec((1, 1, BATCH), lambda i: (i, 0, 0)),
        out_shape=jax.ShapeDtypeStruct((NUM_CLASSES, 1, BATCH), jnp.float32),
    )(h)


def kernel(x, w0, w1, w2, w3, ia0, ib0, ia1, ib1, ia2, ib2, ia3, ib3):
    t = x.reshape(BATCH, IN_SIZE).T
    ws = jnp.stack([w0, w1, w2, w3]).reshape(4, NEURONS // 8, 128)
    coefs = _coef_call(ws)
    oi3d = jnp.arange(NEURONS, dtype=jnp.int32).reshape(NW, NCHUNK, CHUNK)
    for l, (ia, ib) in enumerate([(ia0, ib0), (ia1, ib1), (ia2, ib2), (ia3, ib3)]):
        ia3d = ia.reshape(NW, NCHUNK, CHUNK)
        ib3d = ib.reshape(NW, NCHUNK, CHUNK)
        cl = coefs[l].reshape(NW, 1, 16 * NPW)
        t = _sc_layer(t, ia3d, ib3d, oi3d, cl)
    y = _gsum_call(t)
    return y.reshape(NUM_CLASSES, BATCH).T
